# baseline (speedup 1.0000x reference)
import dataclasses

import jax
import jax.numpy as jnp
from jax import lax
from jax.experimental import pallas as pl
from jax.experimental.pallas import tpu as pltpu
from jax.experimental.pallas import tpu_sc as plsc

BATCH = 16384
F = 16
LANES = 128
NC = 2
NS = 16
NW = NC * NS
RPW = BATCH // NW
GRP = 8
PAIRS = RPW // (2 * GRP)
HPAIRS = PAIRS // 2
SP_BLOCKS = 460
OVF = SP_BLOCKS * LANES
IBLK_PER_TILE = (SP_BLOCKS + NS - 1) // NS
SP_WORDS = SP_BLOCKS * LANES * F

_DNUMS = lax.GatherDimensionNumbers(
    offset_dims=(), collapsed_slice_dims=(0,), start_index_map=(0,))


def _bcast_lane(v, e):
    idx = jnp.full((F, 1), e, jnp.int32)
    return lax.gather(v, idx, dimension_numbers=_DNUMS, slice_sizes=(1,),
                      mode=lax.GatherScatterMode.PROMISE_IN_BOUNDS)


def _gmf_sc(user2d, item2d, ue_t, ie_t, params, dummy):
    mesh = plsc.VectorSubcoreMesh(core_axis_name="c", subcore_axis_name="s")
    cp = pltpu.CompilerParams()
    if "needs_layout_passes" in pltpu.CompilerParams.__dataclass_fields__:
        cp = dataclasses.replace(cp, needs_layout_passes=False)

    @pl.kernel(
        compiler_params=cp,
        out_type=jax.ShapeDtypeStruct((BATCH,), jnp.float32),
        mesh=mesh,
        scratch_types=[
            pltpu.VMEM((RPW,), jnp.int32),
            pltpu.VMEM((RPW,), jnp.int32),
            pltpu.VMEM((F,), jnp.int32),
            pltpu.VMEM((GRP, F, LANES), jnp.float32),
            pltpu.VMEM((GRP, F, LANES), jnp.float32),
            pltpu.VMEM((GRP, F, LANES), jnp.float32),
            pltpu.VMEM((GRP, F, LANES), jnp.float32),
            pltpu.VMEM((F, LANES), jnp.float32),
            pltpu.VMEM((F * LANES,), jnp.float32),
            pltpu.VMEM((2 * GRP * F,), jnp.float32),
            pltpu.VMEM((2 * GRP * F,), jnp.float32),
            pltpu.VMEM_SHARED((SP_WORDS,), jnp.float32),
            pltpu.VMEM((F,), jnp.float32),
            pltpu.VMEM((RPW,), jnp.float32),
            pltpu.VMEM((2, F), jnp.float32),
            pltpu.SemaphoreType.DMA,
            pltpu.SemaphoreType.DMA,
            pltpu.SemaphoreType.DMA,
            pltpu.SemaphoreType.DMA,
        ],
    )
    def k(user_hbm, item_hbm, ue_hbm, ie_hbm, par_hbm, dummy_hbm, out_hbm,
          u_idx, i_idx, fidx_v, ublk0, ublk1, iblk0, iblk1, tA, tB1d,
          irows0, irows1, sp_items, accv, out_v, par_v,
          sem0, sem1, semS0, semS1):
        wid = lax.axis_index("s") * NC + lax.axis_index("c")
        sid = lax.axis_index("s")
        pltpu.sync_copy(user_hbm.at[wid], u_idx)
        pltpu.sync_copy(item_hbm.at[wid], i_idx)
        pltpu.sync_copy(par_hbm, par_v)
        lanes = lax.iota(jnp.int32, F)
        fidx_v[...] = lanes

        ubufs = (ublk0, ublk1)
        ibufs = (iblk0, iblk1)
        sems = (sem0, sem1)
        irowsb = (irows0, irows1)
        semsS = (semS0, semS1)

        def fire(pair, grp, buf):
            base = pl.multiple_of(pair * 2 * GRP, 2 * GRP)
            ub = u_idx[pl.ds(base, F)] & ~(LANES - 1)
            i16 = i_idx[pl.ds(base, F)]
            ib = i16 & ~(LANES - 1)
            for e in range(grp * GRP, (grp + 1) * GRP):
                mask = lanes == e
                bu = pl.multiple_of(
                    jnp.sum(jnp.where(mask, ub, 0)), LANES)
                pltpu.async_copy(ue_hbm.at[fidx_v, pl.ds(bu, LANES)],
                                 ubufs[buf].at[e - grp * GRP], sems[buf])
                it = jnp.sum(jnp.where(mask, i16, 0))

                @pl.when(it >= OVF)
                def _():
                    bi = pl.multiple_of(
                        jnp.sum(jnp.where(mask, ib, 0)), LANES)
                    pltpu.async_copy(ie_hbm.at[fidx_v, pl.ds(bi, LANES)],
                                     ibufs[buf].at[e - grp * GRP], sems[buf])

        def drain(pair, grp, buf):
            pltpu.make_async_copy(dummy_hbm, ubufs[buf], sems[buf]).wait()
            base = pl.multiple_of(pair * 2 * GRP, 2 * GRP)
            i16 = i_idx[pl.ds(base, F)]
            for e in range(grp * GRP, (grp + 1) * GRP):
                it = jnp.sum(jnp.where(lanes == e, i16, 0))

                @pl.when(it >= OVF)
                def _():
                    pltpu.make_async_copy(
                        dummy_hbm.at[0], ibufs[buf].at[e - grp * GRP],
                        sems[buf]).wait()

        def fire_items(pair, w):
            base = pl.multiple_of(pair * 2 * GRP, 2 * GRP)
            i16 = i_idx[pl.ds(base, F)]
            for e in range(2 * GRP):
                it = jnp.sum(jnp.where(lanes == e, i16, 0))

                @pl.when(it < OVF)
                def _():
                    soff = pl.multiple_of(it * F, F)
                    pltpu.async_copy(sp_items.at[pl.ds(soff, F)],
                                     irowsb[w].at[pl.ds(e * F, F)], semsS[w])

        def drain_items(pair, w):
            base = pl.multiple_of(pair * 2 * GRP, 2 * GRP)
            i16 = i_idx[pl.ds(base, F)]
            for e in range(2 * GRP):
                it = jnp.sum(jnp.where(lanes == e, i16, 0))

                @pl.when(it < OVF)
                def _():
                    pltpu.make_async_copy(
                        sp_items.at[pl.ds(0, F)],
                        irowsb[w].at[pl.ds(e * F, F)], semsS[w]).wait()

        fire(0, 0, 0)
        fire(0, 1, 1)

        @pl.loop(0, IBLK_PER_TILE)
        def _(j):
            blk = sid * IBLK_PER_TILE + j

            @pl.when(blk < SP_BLOCKS)
            def _():
                boff = pl.multiple_of(blk * LANES, LANES)
                pltpu.async_copy(ie_hbm.at[fidx_v, pl.ds(boff, LANES)],
                                 tA, semS0).wait()
                for j2 in range(LANES):
                    tB1d[pl.ds(j2 * F, F)] = plsc.load_gather(
                        tA, [lanes, jnp.full((F,), j2, jnp.int32)])
                soff = pl.multiple_of(blk * LANES * F, LANES * F)
                pltpu.sync_copy(tB1d, sp_items.at[pl.ds(soff, LANES * F)])

        plsc.subcore_barrier()

        wvec = par_v[0]
        bvec = par_v[1]
        accv[...] = bvec

        def compute(pair, grp, buf, w):
            base = pl.multiple_of(pair * 2 * GRP, 2 * GRP)
            ulu = u_idx[pl.ds(base, F)] & (LANES - 1)
            i16 = i_idx[pl.ds(base, F)]
            a = accv[...]
            for e in range(grp * GRP, (grp + 1) * GRP):
                ucol = plsc.load_gather(
                    ubufs[buf].at[e - grp * GRP], [lanes, _bcast_lane(ulu, e)])
                it_vec = _bcast_lane(i16, e)
                icol_hbm = plsc.load_gather(
                    ibufs[buf].at[e - grp * GRP],
                    [lanes, it_vec & (LANES - 1)])
                icol = jnp.where(it_vec >= OVF, icol_hbm,
                                 irowsb[w][pl.ds(e * F, F)])
                s = jnp.sum(ucol * icol * wvec)
                a = a + jnp.where(lanes == e, s, 0.0)
            accv[...] = a

        fire_items(0, 0)

        @pl.loop(0, HPAIRS)
        def _(qq):
            pa = qq * 2
            pb = pa + 1

            fire_items(pb, 1)
            drain(pa, 0, 0)
            drain_items(pa, 0)
            compute(pa, 0, 0, 0)
            fire(pb, 0, 0)
            drain(pa, 1, 1)
            compute(pa, 1, 1, 0)
            fire(pb, 1, 1)
            basea = pl.multiple_of(pa * F, F)
            out_v[pl.ds(basea, F)] = accv[...]
            accv[...] = bvec

            @pl.when(qq < HPAIRS - 1)
            def _():
                fire_items(pa + 2, 0)

            drain(pb, 0, 0)
            drain_items(pb, 1)
            compute(pb, 0, 0, 1)

            @pl.when(qq < HPAIRS - 1)
            def _():
                fire(pa + 2, 0, 0)

            drain(pb, 1, 1)
            compute(pb, 1, 1, 1)

            @pl.when(qq < HPAIRS - 1)
            def _():
                fire(pa + 2, 1, 1)

            baseb = pl.multiple_of(pb * F, F)
            out_v[pl.ds(baseb, F)] = accv[...]
            accv[...] = bvec

        pltpu.sync_copy(out_v, out_hbm.at[pl.ds(wid * RPW, RPW)])

    return k(user2d, item2d, ue_t, ie_t, params, dummy)


@jax.jit
def kernel(user, item, user_emb, item_emb, W, b):
    user2d = user.astype(jnp.int32).reshape(NW, RPW)
    item2d = item.astype(jnp.int32).reshape(NW, RPW)
    ue_t = user_emb.T
    ie_t = item_emb.T
    params = jnp.concatenate(
        [W.reshape(1, F), jnp.broadcast_to(b.reshape(1, 1), (1, F))], axis=0)
    dummy = jnp.zeros((GRP, F, LANES), jnp.float32)
    return _gmf_sc(user2d, item2d, ue_t, ie_t, params, dummy)


# speedup vs baseline: 1.8498x; 1.8498x over previous
import dataclasses

import jax
import jax.numpy as jnp
from jax import lax
from jax.experimental import pallas as pl
from jax.experimental.pallas import tpu as pltpu
from jax.experimental.pallas import tpu_sc as plsc

BATCH = 16384
F = 16
LANES = 128
NC = 2
NS = 16
NW = NC * NS
RPW = BATCH // NW
GRP = 8
PAIRS = RPW // (2 * GRP)

_DNUMS = lax.GatherDimensionNumbers(
    offset_dims=(), collapsed_slice_dims=(0,), start_index_map=(0,))


def _bcast_lane(v, e):
    idx = jnp.full((F, 1), e, jnp.int32)
    return lax.gather(v, idx, dimension_numbers=_DNUMS, slice_sizes=(1,),
                      mode=lax.GatherScatterMode.PROMISE_IN_BOUNDS)


def _gmf_sc(user2d, item2d, ue_t, ie_t, params, dummy):
    mesh = plsc.VectorSubcoreMesh(core_axis_name="c", subcore_axis_name="s")
    cp = pltpu.CompilerParams()
    if "needs_layout_passes" in pltpu.CompilerParams.__dataclass_fields__:
        cp = dataclasses.replace(cp, needs_layout_passes=False)

    @pl.kernel(
        compiler_params=cp,
        out_type=jax.ShapeDtypeStruct((BATCH,), jnp.float32),
        mesh=mesh,
        scratch_types=[
            pltpu.VMEM((RPW,), jnp.int32),
            pltpu.VMEM((RPW,), jnp.int32),
            pltpu.VMEM((F,), jnp.int32),
            pltpu.VMEM((GRP, F, LANES), jnp.float32),
            pltpu.VMEM((GRP, F, LANES), jnp.float32),
            pltpu.VMEM((GRP, F, LANES), jnp.float32),
            pltpu.VMEM((GRP, F, LANES), jnp.float32),
            pltpu.VMEM((F,), jnp.float32),
            pltpu.VMEM((RPW,), jnp.float32),
            pltpu.VMEM((2, F), jnp.float32),
            pltpu.SemaphoreType.DMA,
            pltpu.SemaphoreType.DMA,
        ],
    )
    def k(user_hbm, item_hbm, ue_hbm, ie_hbm, par_hbm, dummy_hbm, out_hbm,
          u_idx, i_idx, fidx_v, ublk0, ublk1, iblk0, iblk1, accv, out_v,
          par_v, sem0, sem1):
        wid = lax.axis_index("s") * NC + lax.axis_index("c")
        pltpu.sync_copy(user_hbm.at[wid], u_idx)
        pltpu.sync_copy(item_hbm.at[wid], i_idx)
        pltpu.sync_copy(par_hbm, par_v)
        lanes = lax.iota(jnp.int32, F)
        fidx_v[...] = lanes

        ubufs = (ublk0, ublk1)
        ibufs = (iblk0, iblk1)
        sems = (sem0, sem1)

        def fire(pair, grp, buf):
            base = pl.multiple_of(pair * 2 * GRP, 2 * GRP)
            ub = u_idx[pl.ds(base, F)] & ~(LANES - 1)
            ib = i_idx[pl.ds(base, F)] & ~(LANES - 1)
            for e in range(grp * GRP, (grp + 1) * GRP):
                mask = lanes == e
                bu = pl.multiple_of(
                    jnp.sum(jnp.where(mask, ub, 0)), LANES)
                bi = pl.multiple_of(
                    jnp.sum(jnp.where(mask, ib, 0)), LANES)
                pltpu.async_copy(ue_hbm.at[fidx_v, pl.ds(bu, LANES)],
                                 ubufs[buf].at[e - grp * GRP], sems[buf])
                pltpu.async_copy(ie_hbm.at[fidx_v, pl.ds(bi, LANES)],
                                 ibufs[buf].at[e - grp * GRP], sems[buf])

        def drain(buf):
            pltpu.make_async_copy(dummy_hbm, ubufs[buf], sems[buf]).wait()
            pltpu.make_async_copy(dummy_hbm, ibufs[buf], sems[buf]).wait()

        wvec = par_v[0]
        bvec = par_v[1]
        accv[...] = bvec

        def compute(pair, grp, buf):
            base = pl.multiple_of(pair * 2 * GRP, 2 * GRP)
            ulu = u_idx[pl.ds(base, F)] & (LANES - 1)
            uli = i_idx[pl.ds(base, F)] & (LANES - 1)
            a = accv[...]
            for e in range(grp * GRP, (grp + 1) * GRP):
                ucol = plsc.load_gather(
                    ubufs[buf].at[e - grp * GRP], [lanes, _bcast_lane(ulu, e)])
                icol = plsc.load_gather(
                    ibufs[buf].at[e - grp * GRP], [lanes, _bcast_lane(uli, e)])
                s = jnp.sum(ucol * icol * wvec)
                a = a + jnp.where(lanes == e, s, 0.0)
            accv[...] = a

        fire(0, 0, 0)
        fire(0, 1, 1)

        @pl.loop(0, PAIRS)
        def _(kk):
            drain(0)
            compute(kk, 0, 0)

            @pl.when(kk < PAIRS - 1)
            def _():
                fire(kk + 1, 0, 0)

            drain(1)
            compute(kk, 1, 1)

            @pl.when(kk < PAIRS - 1)
            def _():
                fire(kk + 1, 1, 1)

            base = pl.multiple_of(kk * F, F)
            out_v[pl.ds(base, F)] = accv[...]
            accv[...] = bvec

        pltpu.sync_copy(out_v, out_hbm.at[pl.ds(wid * RPW, RPW)])

    return k(user2d, item2d, ue_t, ie_t, params, dummy)


@jax.jit
def kernel(user, item, user_emb, item_emb, W, b):
    user2d = user.astype(jnp.int32).reshape(NW, RPW)
    item2d = item.astype(jnp.int32).reshape(NW, RPW)
    ue_t = user_emb.T
    ie_t = item_emb.T
    params = jnp.concatenate(
        [W.reshape(1, F), jnp.broadcast_to(b.reshape(1, 1), (1, F))], axis=0)
    dummy = jnp.zeros((GRP, F, LANES), jnp.float32)
    return _gmf_sc(user2d, item2d, ue_t, ie_t, params, dummy)
